# baseline (device time: 57287 ns/iter reference)
import jax
import jax.numpy as jnp
from jax import lax
from jax.experimental import pallas as pl
from jax.experimental.pallas import tpu as pltpu

N_DEV = 8
SQ = 1024
SKV = 1024
DH = 128
H_PER = 8
D_MODEL = 1024
SCALE = 0.08838834764831843
WINDOW = 128
QT = 256
KW = 512
CHUNK = SQ // N_DEV
N_TILE = SQ // QT
BF = jnp.bfloat16


def _body(x_ref, wq_hbm, k_ref, v_ref, wo_hbm, out_ref,
          wq_f32, wo_f32, wqb, wob, q_full,
          ctx_ref, p_ref, bc_src, rs_buf, bc_buf,
          ld_sem, sc_send, sc_recv, bc_send, bc_recv):
    my = lax.axis_index("i")

    cp_wq = pltpu.make_async_copy(
        wq_hbm.at[:, pl.ds(pl.multiple_of(my * D_MODEL, 128), D_MODEL)],
        wq_f32, ld_sem.at[0])
    cp_wq.start()
    cp_wo = pltpu.make_async_copy(
        wo_hbm.at[pl.ds(pl.multiple_of(my * D_MODEL, 128), D_MODEL), :],
        wo_f32, ld_sem.at[1])
    cp_wo.start()

    barrier = pltpu.get_barrier_semaphore()
    for p in range(N_DEV):
        @pl.when(p != my)
        def _():
            pl.semaphore_signal(barrier, inc=1, device_id=(p,),
                                device_id_type=pl.DeviceIdType.MESH)
    pl.semaphore_wait(barrier, N_DEV - 1)

    cp_wq.wait()
    wqb[...] = wq_f32[...].astype(BF)
    q_full[...] = jnp.dot(x_ref[...], wqb[...],
                          preferred_element_type=jnp.float32).astype(BF)
    cp_wo.wait()
    wob[...] = wo_f32[...].astype(BF)

    my_tile = lax.div(my, 2)
    for k in range(N_TILE):
        t = lax.rem(my_tile + 1 + k, N_TILE)
        q0 = t * QT
        rows = pl.ds(pl.multiple_of(q0, 128), QT)
        q_t = q_full[rows, :]
        kstart = jnp.clip(q0 - WINDOW, 0, SKV - KW)
        qi = q0 + lax.broadcasted_iota(jnp.int32, (QT, KW), 0)
        ki = kstart + lax.broadcasted_iota(jnp.int32, (QT, KW), 1)
        mask = jnp.abs(qi - ki) <= WINDOW
        krows = pl.ds(pl.multiple_of(kstart, 128), KW)
        for h in range(H_PER):
            cols = slice(h * DH, (h + 1) * DH)
            scores = lax.dot_general(
                q_t[:, cols], k_ref[krows, cols],
                (((1,), (1,)), ((), ())),
                preferred_element_type=jnp.float32) * SCALE
            w = jnp.exp(jnp.where(mask, scores, -1e9))
            recip = 1.0 / jnp.sum(w, axis=1, keepdims=True)
            ctx_ref[rows, cols] = (jnp.dot(
                w.astype(BF), v_ref[krows, cols],
                preferred_element_type=jnp.float32) * recip).astype(BF)
        p_ref[rows, :] = jnp.dot(ctx_ref[rows, :], wob[...],
                                 preferred_element_type=jnp.float32
                                 ).astype(BF)
        for j in range(2):
            c = 2 * t + j
            @pl.when(c != my)
            def _():
                pltpu.make_async_remote_copy(
                    src_ref=p_ref.at[pl.ds(c * CHUNK, CHUNK), :],
                    dst_ref=rs_buf.at[my],
                    send_sem=sc_send.at[2 * k + j],
                    recv_sem=sc_recv.at[my],
                    device_id=(c,),
                    device_id_type=pl.DeviceIdType.MESH,
                ).start()

    for p in range(N_DEV):
        @pl.when(p != my)
        def _():
            pltpu.make_async_remote_copy(
                src_ref=rs_buf.at[p], dst_ref=rs_buf.at[p],
                send_sem=sc_send.at[0], recv_sem=sc_recv.at[p],
                device_id=(my,), device_id_type=pl.DeviceIdType.MESH,
            ).wait_recv()
    for k in range(N_TILE):
        t = lax.rem(my_tile + 1 + k, N_TILE)
        for j in range(2):
            c = 2 * t + j
            @pl.when(c != my)
            def _():
                pltpu.make_async_remote_copy(
                    src_ref=p_ref.at[pl.ds(c * CHUNK, CHUNK), :],
                    dst_ref=rs_buf.at[my],
                    send_sem=sc_send.at[2 * k + j],
                    recv_sem=sc_recv.at[my],
                    device_id=(c,), device_id_type=pl.DeviceIdType.MESH,
                ).wait_send()

    own = p_ref[pl.ds(my * CHUNK, CHUNK), :]
    red = jnp.zeros((CHUNK, D_MODEL), jnp.float32)
    for j in range(N_DEV):
        red = red + jnp.where(my == j, own, rs_buf[j]).astype(jnp.float32)
    out_ref[0, pl.ds(my * CHUNK, CHUNK), :] = red
    bc_src[...] = red.astype(BF)

    for q in range(N_DEV):
        @pl.when(q != my)
        def _():
            pltpu.make_async_remote_copy(
                src_ref=bc_src,
                dst_ref=bc_buf.at[my],
                send_sem=bc_send.at[q],
                recv_sem=bc_recv.at[my],
                device_id=(q,),
                device_id_type=pl.DeviceIdType.MESH,
            ).start()
    for p in range(N_DEV):
        @pl.when(p != my)
        def _():
            pltpu.make_async_remote_copy(
                src_ref=bc_src, dst_ref=bc_buf.at[p],
                send_sem=bc_send.at[p], recv_sem=bc_recv.at[p],
                device_id=(my,), device_id_type=pl.DeviceIdType.MESH,
            ).wait_recv()
            out_ref[0, pl.ds(p * CHUNK, CHUNK), :] = (
                bc_buf[p].astype(jnp.float32))
    for q in range(N_DEV):
        @pl.when(q != my)
        def _():
            pltpu.make_async_remote_copy(
                src_ref=bc_src, dst_ref=bc_buf.at[my],
                send_sem=bc_send.at[q], recv_sem=bc_recv.at[my],
                device_id=(q,), device_id_type=pl.DeviceIdType.MESH,
            ).wait_send()


def kernel(x, Wq, K_ext, V_ext, Wo):
    xb = x[0].astype(BF)
    kb = K_ext[0].reshape(SKV, H_PER * DH).astype(BF)
    vb = V_ext[0].reshape(SKV, H_PER * DH).astype(BF)

    return pl.pallas_call(
        _body,
        out_shape=jax.ShapeDtypeStruct((1, SQ, D_MODEL), jnp.float32),
        in_specs=[
            pl.BlockSpec(memory_space=pltpu.VMEM),
            pl.BlockSpec(memory_space=pl.ANY),
            pl.BlockSpec(memory_space=pltpu.VMEM),
            pl.BlockSpec(memory_space=pltpu.VMEM),
            pl.BlockSpec(memory_space=pl.ANY),
        ],
        out_specs=pl.BlockSpec(memory_space=pltpu.VMEM),
        scratch_shapes=[
            pltpu.VMEM((D_MODEL, H_PER * DH), jnp.float32),
            pltpu.VMEM((H_PER * DH, D_MODEL), jnp.float32),
            pltpu.VMEM((D_MODEL, H_PER * DH), BF),
            pltpu.VMEM((H_PER * DH, D_MODEL), BF),
            pltpu.VMEM((SQ, H_PER * DH), BF),
            pltpu.VMEM((SQ, H_PER * DH), BF),
            pltpu.VMEM((SQ, D_MODEL), BF),
            pltpu.VMEM((CHUNK, D_MODEL), BF),
            pltpu.VMEM((N_DEV, CHUNK, D_MODEL), BF),
            pltpu.VMEM((N_DEV, CHUNK, D_MODEL), BF),
            pltpu.SemaphoreType.DMA((2,)),
            pltpu.SemaphoreType.DMA((N_DEV,)),
            pltpu.SemaphoreType.DMA((N_DEV,)),
            pltpu.SemaphoreType.DMA((N_DEV,)),
            pltpu.SemaphoreType.DMA((N_DEV,)),
        ],
        compiler_params=pltpu.CompilerParams(
            collective_id=0, vmem_limit_bytes=56 * 1024 * 1024),
    )(xb, Wq, kb, vb, Wo)


# device time: 46594 ns/iter; 1.2295x vs baseline; 1.2295x over previous
import jax
import jax.numpy as jnp
from jax import lax
from jax.experimental import pallas as pl
from jax.experimental.pallas import tpu as pltpu

N_DEV = 8
SQ = 1024
SKV = 1024
DH = 128
H_PER = 8
D_MODEL = 1024
SCALE = 0.08838834764831843
WINDOW = 128
QT = 256
KW = 512
CHUNK = SQ // N_DEV
N_TILE = SQ // QT
BF = jnp.bfloat16


def _body(x_ref, wq_hbm, k_ref, v_ref, wo_hbm, out_ref,
          wq_f32, wo_f32, xb, wqb, wob, kb, vb,
          ctx_ref, p_ref, bc_src, rs_buf, bc_buf,
          ld_sem, sc_send, sc_recv, bc_send, bc_recv):
    my = lax.axis_index("i")

    cp_wq = pltpu.make_async_copy(
        wq_hbm.at[:, pl.ds(pl.multiple_of(my * D_MODEL, 128), D_MODEL)],
        wq_f32, ld_sem.at[0])
    cp_wq.start()
    cp_wo = pltpu.make_async_copy(
        wo_hbm.at[pl.ds(pl.multiple_of(my * D_MODEL, 128), D_MODEL), :],
        wo_f32, ld_sem.at[1])
    cp_wo.start()

    barrier = pltpu.get_barrier_semaphore()
    for p in range(N_DEV):
        @pl.when(p != my)
        def _():
            pl.semaphore_signal(barrier, inc=1, device_id=(p,),
                                device_id_type=pl.DeviceIdType.MESH)
    pl.semaphore_wait(barrier, N_DEV - 1)

    xb[...] = x_ref[...].astype(BF)
    kb[...] = k_ref[...].astype(BF)
    vb[...] = v_ref[...].astype(BF)
    cp_wq.wait()
    wqb[...] = wq_f32[...].astype(BF)
    cp_wo.wait()
    wob[...] = wo_f32[...].astype(BF)

    my_tile = lax.div(my, 2)
    for k in range(N_TILE):
        t = lax.rem(my_tile + 1 + k, N_TILE)
        q0 = t * QT
        rows = pl.ds(pl.multiple_of(q0, 128), QT)
        q_t = jnp.dot(xb[rows, :], wqb[...],
                      preferred_element_type=jnp.float32)
        kstart = jnp.clip(q0 - WINDOW, 0, SKV - KW)
        qi = q0 + lax.broadcasted_iota(jnp.int32, (QT, KW), 0)
        ki = kstart + lax.broadcasted_iota(jnp.int32, (QT, KW), 1)
        mask = jnp.abs(qi - ki) <= WINDOW
        krows = pl.ds(pl.multiple_of(kstart, 128), KW)
        for h in range(H_PER):
            cols = slice(h * DH, (h + 1) * DH)
            scores = lax.dot_general(
                q_t[:, cols].astype(BF), kb[krows, cols],
                (((1,), (1,)), ((), ())),
                preferred_element_type=jnp.float32) * SCALE
            w = jnp.exp(jnp.where(mask, scores, -1e9))
            recip = 1.0 / jnp.sum(w, axis=1, keepdims=True)
            ctx_ref[rows, cols] = (jnp.dot(
                w.astype(BF), vb[krows, cols],
                preferred_element_type=jnp.float32) * recip).astype(BF)
        p_ref[rows, :] = jnp.dot(ctx_ref[rows, :], wob[...],
                                 preferred_element_type=jnp.float32
                                 ).astype(BF)
        for j in range(2):
            c = 2 * t + j
            @pl.when(c != my)
            def _():
                pltpu.make_async_remote_copy(
                    src_ref=p_ref.at[pl.ds(c * CHUNK, CHUNK), :],
                    dst_ref=rs_buf.at[my],
                    send_sem=sc_send.at[2 * k + j],
                    recv_sem=sc_recv.at[my],
                    device_id=(c,),
                    device_id_type=pl.DeviceIdType.MESH,
                ).start()

    for p in range(N_DEV):
        @pl.when(p != my)
        def _():
            pltpu.make_async_remote_copy(
                src_ref=rs_buf.at[p], dst_ref=rs_buf.at[p],
                send_sem=sc_send.at[0], recv_sem=sc_recv.at[p],
                device_id=(my,), device_id_type=pl.DeviceIdType.MESH,
            ).wait_recv()
    for k in range(N_TILE):
        t = lax.rem(my_tile + 1 + k, N_TILE)
        for j in range(2):
            c = 2 * t + j
            @pl.when(c != my)
            def _():
                pltpu.make_async_remote_copy(
                    src_ref=p_ref.at[pl.ds(c * CHUNK, CHUNK), :],
                    dst_ref=rs_buf.at[my],
                    send_sem=sc_send.at[2 * k + j],
                    recv_sem=sc_recv.at[my],
                    device_id=(c,), device_id_type=pl.DeviceIdType.MESH,
                ).wait_send()

    own = p_ref[pl.ds(my * CHUNK, CHUNK), :]
    red = jnp.zeros((CHUNK, D_MODEL), jnp.float32)
    for j in range(N_DEV):
        red = red + jnp.where(my == j, own, rs_buf[j]).astype(jnp.float32)
    out_ref[pl.ds(my * CHUNK, CHUNK), :] = red
    bc_src[...] = red.astype(BF)

    for q in range(N_DEV):
        @pl.when(q != my)
        def _():
            pltpu.make_async_remote_copy(
                src_ref=bc_src,
                dst_ref=bc_buf.at[my],
                send_sem=bc_send.at[q],
                recv_sem=bc_recv.at[my],
                device_id=(q,),
                device_id_type=pl.DeviceIdType.MESH,
            ).start()
    for p in range(N_DEV):
        @pl.when(p != my)
        def _():
            pltpu.make_async_remote_copy(
                src_ref=bc_src, dst_ref=bc_buf.at[p],
                send_sem=bc_send.at[p], recv_sem=bc_recv.at[p],
                device_id=(my,), device_id_type=pl.DeviceIdType.MESH,
            ).wait_recv()
            out_ref[pl.ds(p * CHUNK, CHUNK), :] = (
                bc_buf[p].astype(jnp.float32))
    for q in range(N_DEV):
        @pl.when(q != my)
        def _():
            pltpu.make_async_remote_copy(
                src_ref=bc_src, dst_ref=bc_buf.at[my],
                send_sem=bc_send.at[q], recv_sem=bc_recv.at[my],
                device_id=(q,), device_id_type=pl.DeviceIdType.MESH,
            ).wait_send()


def kernel(x, Wq, K_ext, V_ext, Wo):
    x2 = x[0]
    kv_k = K_ext[0].reshape(SKV, H_PER * DH)
    kv_v = V_ext[0].reshape(SKV, H_PER * DH)

    out = pl.pallas_call(
        _body,
        out_shape=jax.ShapeDtypeStruct((SQ, D_MODEL), jnp.float32),
        in_specs=[
            pl.BlockSpec(memory_space=pltpu.VMEM),
            pl.BlockSpec(memory_space=pl.ANY),
            pl.BlockSpec(memory_space=pltpu.VMEM),
            pl.BlockSpec(memory_space=pltpu.VMEM),
            pl.BlockSpec(memory_space=pl.ANY),
        ],
        out_specs=pl.BlockSpec(memory_space=pltpu.VMEM),
        scratch_shapes=[
            pltpu.VMEM((D_MODEL, H_PER * DH), jnp.float32),
            pltpu.VMEM((H_PER * DH, D_MODEL), jnp.float32),
            pltpu.VMEM((SQ, D_MODEL), BF),
            pltpu.VMEM((D_MODEL, H_PER * DH), BF),
            pltpu.VMEM((H_PER * DH, D_MODEL), BF),
            pltpu.VMEM((SKV, H_PER * DH), BF),
            pltpu.VMEM((SKV, H_PER * DH), BF),
            pltpu.VMEM((SQ, H_PER * DH), BF),
            pltpu.VMEM((SQ, D_MODEL), BF),
            pltpu.VMEM((CHUNK, D_MODEL), BF),
            pltpu.VMEM((N_DEV, CHUNK, D_MODEL), BF),
            pltpu.VMEM((N_DEV, CHUNK, D_MODEL), BF),
            pltpu.SemaphoreType.DMA((2,)),
            pltpu.SemaphoreType.DMA((N_DEV,)),
            pltpu.SemaphoreType.DMA((N_DEV,)),
            pltpu.SemaphoreType.DMA((N_DEV,)),
            pltpu.SemaphoreType.DMA((N_DEV,)),
        ],
        compiler_params=pltpu.CompilerParams(collective_id=0),
    )(x2, Wq, kv_k, kv_v, Wo)
    return out[None]
